# Initial kernel scaffold; baseline (speedup 1.0000x reference)
import jax
import jax.numpy as jnp
from jax.experimental import pallas as pl


def kernel(x, edge_index, Ws, bs):
    raise NotImplementedError("write your pallas kernel here")


# speedup vs baseline: 10.5719x; 10.5719x over previous
import functools

import jax
import jax.numpy as jnp
from jax import lax
from jax.experimental import pallas as pl
from jax.experimental.pallas import tpu as pltpu
from jax.experimental.pallas import tpu_sc as plsc

N = 10000
D_IN = 128
NCLS = 64
KL = 10
ALPHA = 0.1

NC = 2
NS = 16
NW = NC * NS
C = 128
NPAD = 10240
RPT = NPAD // NS
RCH = 128


def _make_edge_pass(num_chunks: int):
    mesh = plsc.VectorSubcoreMesh(core_axis_name="c", subcore_axis_name="s")

    @functools.partial(
        pl.kernel,
        out_type=jax.ShapeDtypeStruct((NC, NPAD, NCLS), jnp.float32),
        mesh=mesh,
        compiler_params=pltpu.CompilerParams(use_tc_tiling_on_sc=False),
        scratch_types=[
            pltpu.VMEM((C,), jnp.int32),
            pltpu.VMEM((C,), jnp.int32),
            pltpu.VMEM((C, NCLS), jnp.float32),
            pltpu.VMEM_SHARED((NPAD, NCLS), jnp.float32),
            pltpu.SemaphoreType.DMA,
        ],
    )
    def edge_pass(z_hbm, src_hbm, dst_hbm, zeros_hbm, out_hbm,
                  srcb, dstb, rows, acc, sem):
        cid = lax.axis_index("c")
        sid = lax.axis_index("s")
        wid = sid * NC + cid

        def zbody(r, _):
            pltpu.sync_copy(zeros_hbm, acc.at[pl.ds(sid * RPT + r * RCH, RCH)])
            return 0
        lax.fori_loop(0, RPT // RCH, zbody, 0)
        plsc.subcore_barrier()

        def body(ch, _):
            pltpu.sync_copy(src_hbm.at[wid, ch], srcb)
            pltpu.sync_copy(dst_hbm.at[wid, ch], dstb)
            pltpu.async_copy(z_hbm.at[srcb], rows, sem).wait()
            pltpu.sync_copy(rows, acc.at[dstb], add=True)
            return 0
        lax.fori_loop(0, num_chunks, body, 0)
        plsc.subcore_barrier()

        def wbody(r, _):
            off = sid * RPT + r * RCH
            pltpu.sync_copy(acc.at[pl.ds(off, RCH)],
                            out_hbm.at[cid, pl.ds(off, RCH)])
            return 0
        lax.fori_loop(0, RPT // RCH, wbody, 0)

    return edge_pass


def _mlp(x_pad, dinv64, Ws, bs):
    R = 512
    G = NPAD // R

    def body(x_ref, d_ref, *refs):
        w_refs = refs[:KL]
        b_refs = refs[KL:2 * KL]
        x0_ref, z0_ref = refs[2 * KL], refs[2 * KL + 1]
        h = x_ref[...]
        for i in range(KL):
            h = jnp.dot(h, w_refs[i][...], preferred_element_type=jnp.float32)
            h = h + b_refs[i][...]
            if i != KL - 1:
                h = jnp.maximum(h, 0.0)
        x0_ref[...] = h
        z0_ref[...] = d_ref[...] * h

    in_specs = [pl.BlockSpec((R, D_IN), lambda i: (i, 0)),
                pl.BlockSpec((R, NCLS), lambda i: (i, 0))]
    for W in Ws:
        in_specs.append(pl.BlockSpec(W.shape, lambda i: (0, 0)))
    for b in bs:
        in_specs.append(pl.BlockSpec((1, b.shape[0]), lambda i: (0, 0)))
    out_specs = [pl.BlockSpec((R, NCLS), lambda i: (i, 0))] * 2
    out_shape = [jax.ShapeDtypeStruct((NPAD, NCLS), jnp.float32)] * 2
    return pl.pallas_call(
        body, grid=(G,), in_specs=in_specs, out_specs=out_specs,
        out_shape=out_shape,
    )(x_pad, dinv64, *Ws, *[b[None, :] for b in bs])


def _combine(accs, dinv64, z0):
    NR = NPAD * NCLS // 128
    R = 512
    a = accs.reshape(NC, NR, 128)
    d = dinv64.reshape(NR, 128)
    z = z0.reshape(NR, 128)

    def body(a_ref, d_ref, z_ref, o_ref):
        dd = d_ref[...]
        o_ref[...] = ((1.0 - ALPHA) * dd * dd * (a_ref[0] + a_ref[1])
                      + ALPHA * z_ref[...])

    out = pl.pallas_call(
        body, grid=(NR // R,),
        in_specs=[pl.BlockSpec((NC, R, 128), lambda i: (0, i, 0)),
                  pl.BlockSpec((R, 128), lambda i: (i, 0)),
                  pl.BlockSpec((R, 128), lambda i: (i, 0))],
        out_specs=pl.BlockSpec((R, 128), lambda i: (i, 0)),
        out_shape=jax.ShapeDtypeStruct((NR, 128), jnp.float32),
    )(a, d, z)
    return out.reshape(NPAD, NCLS)


def _final(accs, dinv64, x0):
    R = 512

    def body(a_ref, d_ref, x_ref, o_ref):
        h = ((1.0 - ALPHA) * d_ref[...] * (a_ref[0] + a_ref[1])
             + ALPHA * x_ref[...])
        m = jnp.max(h, axis=1, keepdims=True)
        e = jnp.exp(h - m)
        s = jnp.sum(e, axis=1, keepdims=True)
        o_ref[...] = h - m - jnp.log(s)

    return pl.pallas_call(
        body, grid=(NPAD // R,),
        in_specs=[pl.BlockSpec((NC, R, NCLS), lambda i: (0, i, 0)),
                  pl.BlockSpec((R, NCLS), lambda i: (i, 0)),
                  pl.BlockSpec((R, NCLS), lambda i: (i, 0))],
        out_specs=pl.BlockSpec((R, NCLS), lambda i: (i, 0)),
        out_shape=jax.ShapeDtypeStruct((NPAD, NCLS), jnp.float32),
    )(accs, dinv64, x0)


def kernel(x, edge_index, Ws, bs):
    E = edge_index.shape[1]
    etot = E + N
    num_chunks = -(-etot // (NW * C))
    epad = NW * num_chunks * C

    src = edge_index[0].astype(jnp.int32)
    dst = edge_index[1].astype(jnp.int32)
    loop = jnp.arange(N, dtype=jnp.int32)
    fill = jnp.full((epad - etot,), N, dtype=jnp.int32)
    src3 = jnp.concatenate([src, loop, fill]).reshape(NW, num_chunks, C)
    dst3 = jnp.concatenate([dst, loop, fill]).reshape(NW, num_chunks, C)
    zeros_chunk = jnp.zeros((RCH, NCLS), jnp.float32)

    edge_pass = _make_edge_pass(num_chunks)

    ones_z = jnp.zeros((NPAD, NCLS), jnp.float32).at[:N].set(1.0)
    accs = edge_pass(ones_z, src3, dst3, zeros_chunk)
    deg = accs[0, :N, 0] + accs[1, :N, 0]
    dinv = lax.rsqrt(deg)
    dinv_pad = jnp.zeros((NPAD,), jnp.float32).at[:N].set(dinv)
    dinv64 = jnp.broadcast_to(dinv_pad[:, None], (NPAD, NCLS))

    x_pad = jnp.zeros((NPAD, D_IN), jnp.float32).at[:N].set(x)
    x0, z0 = _mlp(x_pad, dinv64, Ws, bs)

    z = z0
    for _ in range(KL - 1):
        accs = edge_pass(z, src3, dst3, zeros_chunk)
        z = _combine(accs, dinv64, z0)
    accs = edge_pass(z, src3, dst3, zeros_chunk)
    out_pad = _final(accs, dinv64, x0)
    return out_pad[:N]
